# Initial kernel scaffold; baseline (speedup 1.0000x reference)
import jax
import jax.numpy as jnp
from jax.experimental import pallas as pl


def kernel(x, raw_x, policy, Wqkv, ys_param, n):
    raise NotImplementedError("write your pallas kernel here")


# speedup vs baseline: 1.0575x; 1.0575x over previous
import jax
import jax.numpy as jnp
from jax.experimental import pallas as pl
from jax.experimental.pallas import tpu as pltpu

_B, _N, _C, _H = 32, 197, 384, 12
_DH = _C // _H
_M = _N - 1
_F32 = jnp.float32
_HI = jax.lax.Precision.HIGHEST


def _eye(n):
    i_io = jax.lax.broadcasted_iota(jnp.int32, (n, n), 0)
    j_io = jax.lax.broadcasted_iota(jnp.int32, (n, n), 1)
    return (i_io == j_io).astype(_F32)


def _col_to_row(col, n):
    return jnp.sum(jnp.broadcast_to(col, (n, n)) * _eye(n), axis=0, keepdims=True)


def _row_to_col(row, n):
    return jnp.sum(jnp.broadcast_to(row, (n, n)) * _eye(n), axis=1, keepdims=True)


def _scores(x, Wqkv):
    Bs, Ns, Cs = x.shape
    scale = _DH ** (-0.5)
    qkv = (x @ Wqkv.T).reshape(Bs, Ns, 3, _H, _DH).transpose(2, 0, 3, 1, 4)
    q, k, v = qkv[0], qkv[1], qkv[2]
    attn = jnp.einsum('bhnd,bhmd->bhnm', q, k) * scale
    attn = jax.nn.softmax(attn, axis=-1)
    v_full = v.transpose(0, 2, 1, 3).reshape(Bs, Ns, Cs)
    v_norm_2 = jnp.linalg.norm(v_full, ord=2, axis=2)
    selection_score = attn[:, :, 0].sum(axis=1) * v_norm_2
    selection_score = selection_score[:, 1:]
    return selection_score / selection_score.sum(axis=1, keepdims=True)


def _rank_kernel(s_ref, ss_ref, sind_ref):
    s_row = s_ref[0]
    s_col = _row_to_col(s_row, _M)

    i_io = jax.lax.broadcasted_iota(jnp.int32, (_M, _M), 0)
    j_io = jax.lax.broadcasted_iota(jnp.int32, (_M, _M), 1)
    si = jnp.broadcast_to(s_col, (_M, _M))
    sj = jnp.broadcast_to(s_row, (_M, _M))
    less = (sj < si) | ((sj == si) & (j_io < i_io))
    rank_col = jnp.sum(less.astype(_F32), axis=1, keepdims=True)
    rank_row = _col_to_row(rank_col, _M)

    perm = (jnp.broadcast_to(rank_row, (_M, _M))
            == i_io.astype(_F32)).astype(_F32)
    ss_col = jnp.sum(perm * sj, axis=1, keepdims=True)
    sind_col = jnp.sum(perm * j_io.astype(_F32), axis=1, keepdims=True)

    ss_ref[0] = _col_to_row(ss_col, _M)
    sind_ref[0] = _col_to_row(sind_col, _M)


def _cdf_kernel(ss_ref, ncdf_ref, t_ref, c_ref):
    ss = ss_ref[:, 0, :]
    eyeM = _eye(_M)
    t_ref[...] = jax.lax.dot_general(
        eyeM, ss, (((1,), (1,)), ((), ())), preferred_element_type=_F32,
        precision=_HI)

    def scan1(j, acc):
        acc = acc + t_ref[pl.ds(j, 1), :]
        c_ref[pl.ds(j, 1), :] = acc
        return acc

    carry = jax.lax.fori_loop(0, 128, scan1, jnp.zeros((1, _B), _F32))

    def scan2(j, acc):
        acc = acc + t_ref[pl.ds(j, 1), :]
        c_ref[pl.ds(j, 1), :] = carry + acc
        return acc

    jax.lax.fori_loop(128, _M, scan2, jnp.zeros((1, _B), _F32))

    cdf = c_ref[...]
    cmin = jnp.min(cdf, axis=0, keepdims=True)
    cmax = jnp.max(cdf, axis=0, keepdims=True)
    ncdf = (cdf - cmin) / (cmax - cmin)
    ncdf_ref[:, 0, :] = jax.lax.dot_general(
        _eye(_B), ncdf, (((1,), (1,)), ((), ())), preferred_element_type=_F32,
        precision=_HI)


def _pick_kernel(ncdf_ref, sind_ref, ys_ref, raw_ref, out_ref):
    ncdf_row = ncdf_ref[0]
    sind_row = sind_ref[0]
    raw = raw_ref[0]

    i_io = jax.lax.broadcasted_iota(jnp.int32, (_M, _M), 0)
    j_io = jax.lax.broadcasted_iota(jnp.int32, (_M, _M), 1)
    ys_col = _row_to_col(ys_ref[0:1, 0:_M], _M)
    d = jnp.abs(jnp.broadcast_to(ys_col, (_M, _M))
                - jnp.broadcast_to(ncdf_row, (_M, _M)))
    dmin = jnp.min(d, axis=1, keepdims=True)
    cand = jnp.where(d == dmin, j_io, _M)
    pick_col = jnp.min(cand, axis=1, keepdims=True).astype(_F32)

    fmask = (j_io.astype(_F32) == jnp.broadcast_to(pick_col, (_M, _M))
             ).astype(_F32)
    final_col = jnp.sum(fmask * jnp.broadcast_to(sind_row, (_M, _M)),
                        axis=1, keepdims=True)

    ind_full = jnp.concatenate(
        [jnp.zeros((1, 1), _F32), final_col + 1.0], axis=0)
    m_io = jax.lax.broadcasted_iota(jnp.int32, (_N, _N), 1).astype(_F32)
    g = (m_io == jnp.broadcast_to(ind_full, (_N, _N))).astype(_F32)
    out_ref[0] = jax.lax.dot_general(
        g, raw, (((1,), (0,)), ((), ())), preferred_element_type=_F32,
        precision=_HI)


def _rank_call(sn):
    return pl.pallas_call(
        _rank_kernel,
        grid=(_B,),
        in_specs=[pl.BlockSpec((1, 1, _M), lambda b: (b, 0, 0))],
        out_specs=[pl.BlockSpec((1, 1, _M), lambda b: (b, 0, 0))] * 2,
        out_shape=[jax.ShapeDtypeStruct((_B, 1, _M), _F32)] * 2,
        compiler_params=pltpu.CompilerParams(
            dimension_semantics=("arbitrary",)),
    )(sn)


def _cdf_call(ss):
    return pl.pallas_call(
        _cdf_kernel,
        grid=(1,),
        in_specs=[pl.BlockSpec((_B, 1, _M), lambda i: (0, 0, 0))],
        out_specs=pl.BlockSpec((_B, 1, _M), lambda i: (0, 0, 0)),
        out_shape=jax.ShapeDtypeStruct((_B, 1, _M), _F32),
        scratch_shapes=[pltpu.VMEM((_M, _B), _F32),
                        pltpu.VMEM((_M, _B), _F32)],
    )(ss)


def _pick_call(ncdf, sind, ys_param, raw_x):
    return pl.pallas_call(
        _pick_kernel,
        grid=(_B,),
        in_specs=[
            pl.BlockSpec((1, 1, _M), lambda b: (b, 0, 0)),
            pl.BlockSpec((1, 1, _M), lambda b: (b, 0, 0)),
            pl.BlockSpec((1, _M), lambda b: (0, 0)),
            pl.BlockSpec((1, _N, _C), lambda b: (b, 0, 0)),
        ],
        out_specs=pl.BlockSpec((1, _N, _C), lambda b: (b, 0, 0)),
        out_shape=jax.ShapeDtypeStruct((_B, _N, _C), _F32),
        compiler_params=pltpu.CompilerParams(
            dimension_semantics=("arbitrary",)),
    )(ncdf, sind, ys_param, raw_x)


def kernel(x, raw_x, policy, Wqkv, ys_param, n):
    sn = _scores(x, Wqkv)
    ss, sind = _rank_call(sn.reshape(_B, 1, _M))
    ncdf = _cdf_call(ss)
    selected = _pick_call(ncdf, sind, ys_param, raw_x)
    return x, selected, policy
